# initial kernel scaffold (unmeasured)
import jax
import jax.numpy as jnp
from jax import lax
from jax.experimental import pallas as pl
from jax.experimental.pallas import tpu as pltpu

N_DEV = 32


def kernel(x, w_mat):
    m_per, k = x.shape
    n_per = w_mat.shape[1]
    m_glob = N_DEV * m_per

    def body(x_ref, w_ref, out_ref, xg_ref, send_sems, recv_sems):
        my = lax.axis_index("i")

        xg_ref[pl.ds(my * m_per, m_per), :] = x_ref[...]

        sends = []
        for d in range(1, N_DEV):
            peer = lax.rem(my + d, N_DEV)
            send = pltpu.make_async_remote_copy(
                src_ref=x_ref,
                dst_ref=xg_ref.at[pl.ds(my * m_per, m_per), :],
                send_sem=send_sems.at[d - 1],
                recv_sem=recv_sems.at[d - 1],
                device_id=(peer,),
                device_id_type=pl.DeviceIdType.MESH,
            )
            send.start()
            sends.append(send)

        for d in range(1, N_DEV):
            origin = lax.rem(my - d + N_DEV, N_DEV)
            recv = pltpu.make_async_remote_copy(
                src_ref=x_ref,
                dst_ref=xg_ref.at[pl.ds(origin * m_per, m_per), :],
                send_sem=send_sems.at[d - 1],
                recv_sem=recv_sems.at[d - 1],
                device_id=(origin,),
                device_id_type=pl.DeviceIdType.MESH,
            )
            recv.wait_recv()

        y = jnp.dot(xg_ref[...], w_ref[...], preferred_element_type=jnp.float32)
        out_ref[...] = y * (1.0 / (1.0 + jnp.exp(-y)))

        for s in sends:
            s.wait_send()

    return pl.pallas_call(
        body,
        out_shape=jax.ShapeDtypeStruct((m_glob, n_per), jnp.float32),
        in_specs=[
            pl.BlockSpec(memory_space=pltpu.VMEM),
            pl.BlockSpec(memory_space=pltpu.VMEM),
        ],
        out_specs=pl.BlockSpec(memory_space=pltpu.VMEM),
        scratch_shapes=[
            pltpu.VMEM((m_glob, k), jnp.float32),
            pltpu.SemaphoreType.DMA((N_DEV - 1,)),
            pltpu.SemaphoreType.DMA((N_DEV - 1,)),
        ],
        compiler_params=pltpu.CompilerParams(collective_id=0),
    )(x, w_mat)


# baseline (device time: 65026 ns/iter reference)
import jax
import jax.numpy as jnp
from jax import lax
from jax.experimental import pallas as pl
from jax.experimental.pallas import tpu as pltpu

N_DEV = 32


def kernel(x, w_mat):
    m_per, k = x.shape
    n_per = w_mat.shape[1]
    m_glob = N_DEV * m_per

    def body(x_ref, w_ref, out_ref, xg_ref, send_sems, recv_sems):
        my = lax.axis_index("i")

        xg_ref[pl.ds(my * m_per, m_per), :] = x_ref[...]

        sends = []
        for d in range(1, N_DEV):
            peer = lax.rem(my + d, N_DEV)
            send = pltpu.make_async_remote_copy(
                src_ref=x_ref,
                dst_ref=xg_ref.at[pl.ds(my * m_per, m_per), :],
                send_sem=send_sems.at[d - 1],
                recv_sem=recv_sems.at[d - 1],
                device_id=(peer,),
                device_id_type=pl.DeviceIdType.MESH,
            )
            send.start()
            sends.append(send)

        for d in range(1, N_DEV):
            origin = lax.rem(my - d + N_DEV, N_DEV)
            recv = pltpu.make_async_remote_copy(
                src_ref=x_ref,
                dst_ref=xg_ref.at[pl.ds(origin * m_per, m_per), :],
                send_sem=send_sems.at[d - 1],
                recv_sem=recv_sems.at[d - 1],
                device_id=(origin,),
                device_id_type=pl.DeviceIdType.MESH,
            )
            recv.wait_recv()

        y = jnp.dot(xg_ref[...], w_ref[...], preferred_element_type=jnp.float32)
        out_ref[...] = y * (1.0 / (1.0 + jnp.exp(-y)))

        for s in sends:
            s.wait_send()

    return pl.pallas_call(
        body,
        out_shape=jax.ShapeDtypeStruct((m_glob, n_per), jnp.float32),
        in_specs=[
            pl.BlockSpec(memory_space=pltpu.VMEM),
            pl.BlockSpec(memory_space=pltpu.VMEM),
        ],
        out_specs=pl.BlockSpec(memory_space=pltpu.VMEM),
        scratch_shapes=[
            pltpu.VMEM((m_glob, k), jnp.float32),
            pltpu.SemaphoreType.DMA((N_DEV - 1,)),
            pltpu.SemaphoreType.DMA((N_DEV - 1,)),
        ],
    )(x, w_mat)


# device time: 52034 ns/iter; 1.2497x vs baseline; 1.2497x over previous
import jax
import jax.numpy as jnp
from jax import lax
from jax.experimental import pallas as pl
from jax.experimental.pallas import tpu as pltpu

N_DEV = 32
NX, NY, NZ = 2, 4, 4


def kernel(x, w_mat):
    m_per, k = x.shape
    n_per = w_mat.shape[1]
    m_glob = N_DEV * m_per

    def body(x_ref, w_ref, out_ref, xg_ref, zr_sems, yr_sems, xr_sems, s_sems):
        my = lax.axis_index("i")
        mz = lax.div(my, 8)
        rem = lax.rem(my, 8)
        myy = lax.div(rem, 2)
        r2 = lax.rem(rem, 2)
        mx = jnp.where(lax.rem(myy, 2) == 0, r2, 1 - r2)

        def log_idx(px, py, pz):
            return pz * 8 + py * 2 + jnp.where(lax.rem(py, 2) == 0, px, 1 - px)

        def rows(o):
            return (pl.ds(o * m_per, m_per), slice(None))

        send_slot = [0]
        sends = []

        def send_chunk(o, target, r_sems, r_slot):
            s = pltpu.make_async_remote_copy(
                src_ref=xg_ref.at[rows(o)],
                dst_ref=xg_ref.at[rows(o)],
                send_sem=s_sems.at[send_slot[0]],
                recv_sem=r_sems.at[r_slot],
                device_id=(target,),
                device_id_type=pl.DeviceIdType.MESH,
            )
            send_slot[0] += 1
            s.start()
            sends.append(s)

        def wait_chunk(o, r_sems, r_slot):
            pltpu.make_async_remote_copy(
                src_ref=xg_ref.at[rows(o)],
                dst_ref=xg_ref.at[rows(o)],
                send_sem=s_sems.at[0],
                recv_sem=r_sems.at[r_slot],
                device_id=(my,),
                device_id_type=pl.DeviceIdType.MESH,
            ).wait_recv()

        xg_ref[rows(my)] = x_ref[...]

        for dz in range(1, NZ):
            t = log_idx(mx, myy, lax.rem(mz + dz, NZ))
            send_chunk(my, t, zr_sems, 3 - dz)
        for dy in range(1, NY):
            t = log_idx(mx, lax.rem(myy + dy, NY), mz)
            send_chunk(my, t, yr_sems, (3 - dy) * NZ + 0)
        send_chunk(my, log_idx(1 - mx, myy, mz), xr_sems, 0)

        for s in range(NZ - 1):
            a = s + 1
            o = log_idx(mx, myy, lax.rem(mz + a, NZ))
            wait_chunk(o, zr_sems, s)
            for dy in range(1, NY):
                t = log_idx(mx, lax.rem(myy + dy, NY), mz)
                send_chunk(o, t, yr_sems, (3 - dy) * NZ + a)
            send_chunk(o, log_idx(1 - mx, myy, mz), xr_sems, 0 * NZ + a)

        for sy in range(NY - 1):
            b = sy + 1
            for a in range(NZ):
                o = log_idx(mx, lax.rem(myy + b, NY), lax.rem(mz + a, NZ))
                wait_chunk(o, yr_sems, sy * NZ + a)
                send_chunk(o, log_idx(1 - mx, myy, mz), xr_sems, b * NZ + a)

        for b in range(NY):
            for a in range(NZ):
                o = log_idx(1 - mx, lax.rem(myy + b, NY), lax.rem(mz + a, NZ))
                wait_chunk(o, xr_sems, b * NZ + a)

        y = jnp.dot(xg_ref[...], w_ref[...], preferred_element_type=jnp.float32)
        out_ref[...] = y * (1.0 / (1.0 + jnp.exp(-y)))

        for s in sends:
            s.wait_send()

    return pl.pallas_call(
        body,
        out_shape=jax.ShapeDtypeStruct((m_glob, n_per), jnp.float32),
        in_specs=[
            pl.BlockSpec(memory_space=pltpu.VMEM),
            pl.BlockSpec(memory_space=pltpu.VMEM),
        ],
        out_specs=pl.BlockSpec(memory_space=pltpu.VMEM),
        scratch_shapes=[
            pltpu.VMEM((m_glob, k), jnp.float32),
            pltpu.SemaphoreType.DMA((NZ - 1,)),
            pltpu.SemaphoreType.DMA(((NY - 1) * NZ,)),
            pltpu.SemaphoreType.DMA((NY * NZ,)),
            pltpu.SemaphoreType.DMA((31,)),
        ],
    )(x, w_mat)
